# baseline (speedup 1.0000x reference)
import functools

import jax
import jax.numpy as jnp
from jax import lax
from jax.experimental import pallas as pl
from jax.experimental.pallas import tpu as pltpu
from jax.experimental.pallas import tpu_sc as plsc

B, S, H = 16, 512, 256
F = 256
MAXLEN = 2048

NC, NS = 2, 16
NW = NC * NS
HALF = MAXLEN // NW * NS
ZROW = B * S
CH = 128
NCHUNK = HALF // CH
LANES = 16


def _regulator_kernel(x_hbm, dur_hbm, sl_hbm, out_hbm, tl_hbm,
                      dur_v, sl_v, idx_v, tl_v, buf0, buf1, zbuf,
                      gsem0, gsem1, osem0, osem1, zsem0, zsem1):
    cid = lax.axis_index("c")
    sid = lax.axis_index("s")
    wid = sid * NC + cid
    b = wid // 2
    half = wid % 2

    pltpu.sync_copy(dur_hbm.at[b], dur_v)
    pltpu.sync_copy(sl_hbm, sl_v)

    lane = lax.iota(jnp.int32, LANES)
    bfill = jnp.full((LANES,), b * S, jnp.int32)
    for j in range(NCHUNK):
        for c in range(CH // LANES):
            idx_v[j, pl.ds(c * LANES, LANES)] = bfill

    zvec = jnp.zeros((LANES,), jnp.float32)

    def zrow(i, _):
        for c in range(H // LANES):
            zbuf[i, pl.ds(c * LANES, LANES)] = zvec
        return 0
    lax.fori_loop(0, CH, zrow, 0)

    out_base = b * MAXLEN
    zsems = (zsem0, zsem1)

    def chunk_start(j):
        return out_base + (2 * j + half) * CH

    for j in range(NCHUNK):
        pltpu.async_copy(
            zbuf, out_hbm.at[pl.ds(chunk_start(j), CH)], zsems[j % 2])

    sl_b = jnp.sum(jnp.where(lane == b, sl_v[...], 0))
    carry = jnp.int32(0)
    for c in range(S // LANES):
        t = c * LANES + lane
        d = dur_v[pl.ds(c * LANES, LANES)]
        d = jnp.where(t < sl_b, d, 0)
        cum = plsc.cumsum(d) + carry
        carry = jnp.max(cum)
        prev = cum - d
        gidx = b * S + t
        for r in range(3):
            pos = prev + r
            chunk = lax.shift_right_logical(pos, 7)
            m = (r < d) & (pos < MAXLEN) & ((chunk & 1) == half)
            plsc.store_scatter(
                idx_v,
                [lax.shift_right_logical(pos, 8), pos & (CH - 1)],
                gidx, mask=m)

    @pl.when(half == 0)
    def _():
        tl_v[...] = jnp.full((LANES,), carry, jnp.int32)
        pltpu.sync_copy(tl_v, tl_hbm.at[b])

    bufs = (buf0, buf1)
    gsems = (gsem0, gsem1)
    osems = (osem0, osem1)

    def is_real(j):
        return (2 * j + half) * CH < carry

    def issue(j):
        p = j % 2

        @pl.when(is_real(j))
        def _():
            pltpu.async_copy(x_hbm.at[idx_v.at[j]], bufs[p], gsems[p])

    def process(j):
        p = j % 2
        lo = (2 * j + half) * CH
        dst = out_hbm.at[pl.ds(chunk_start(j), CH)]
        pltpu.make_async_copy(zbuf, dst, zsems[p]).wait()

        @pl.when(is_real(j))
        def _():
            pltpu.make_async_copy(
                x_hbm.at[idx_v.at[j]], bufs[p], gsems[p]).wait()
            nreal = jnp.clip(carry - lo, 0, CH)

            def ztail(i, _):
                for c in range(H // LANES):
                    bufs[p][i, pl.ds(c * LANES, LANES)] = zvec
                return 0
            lax.fori_loop(nreal, CH, ztail, 0)
            pltpu.async_copy(bufs[p], dst, osems[p])

    def drain_out(j):
        p = j % 2

        @pl.when(is_real(j))
        def _():
            pltpu.make_async_copy(
                bufs[p], out_hbm.at[pl.ds(chunk_start(j), CH)],
                osems[p]).wait()

    for j in range(NCHUNK):
        if j >= 2:
            drain_out(j - 2)
        issue(j)
        if j >= 1:
            process(j - 1)
    process(NCHUNK - 1)
    drain_out(NCHUNK - 2)
    drain_out(NCHUNK - 1)


def _regulate(x2d, durations, src_lens):
    mesh = plsc.VectorSubcoreMesh(
        core_axis_name="c", subcore_axis_name="s",
        num_cores=NC, num_subcores=NS)
    run = functools.partial(
        pl.kernel,
        out_type=(
            jax.ShapeDtypeStruct((B * MAXLEN, H), jnp.float32),
            jax.ShapeDtypeStruct((B, LANES), jnp.int32),
        ),
        mesh=mesh,
        scratch_types=[
            pltpu.VMEM((S,), jnp.int32),
            pltpu.VMEM((LANES,), jnp.int32),
            pltpu.VMEM((NCHUNK, CH), jnp.int32),
            pltpu.VMEM((LANES,), jnp.int32),
            pltpu.VMEM((CH, H), jnp.float32),
            pltpu.VMEM((CH, H), jnp.float32),
            pltpu.VMEM((CH, H), jnp.float32),
            pltpu.SemaphoreType.DMA,
            pltpu.SemaphoreType.DMA,
            pltpu.SemaphoreType.DMA,
            pltpu.SemaphoreType.DMA,
            pltpu.SemaphoreType.DMA,
            pltpu.SemaphoreType.DMA,
        ],
        compiler_params=pltpu.CompilerParams(needs_layout_passes=False),
    )(_regulator_kernel)
    return run(x2d, durations, src_lens)


def _predictor_body(x_ref, m_ref, w1_ref, b1_ref, g1_ref, be1_ref,
                    w2_ref, b2_ref, g2_ref, be2_ref, lw_ref, lb_ref, o_ref):
    def conv(h, w_ref, b):
        z = jnp.zeros((1, F), jnp.float32)
        z0 = jnp.dot(h, w_ref[0], preferred_element_type=jnp.float32)
        z1 = jnp.dot(h, w_ref[1], preferred_element_type=jnp.float32)
        z2 = jnp.dot(h, w_ref[2], preferred_element_type=jnp.float32)
        return (jnp.concatenate([z, z0[:-1]], axis=0) + z1 +
                jnp.concatenate([z2[1:], z], axis=0) + b)

    def layer_norm(h, g, be):
        mu = jnp.mean(h, axis=-1, keepdims=True)
        ctr = h - mu
        v = jnp.mean(ctr * ctr, axis=-1, keepdims=True)
        return ctr / jnp.sqrt(v + 1e-5) * g + be

    def per_batch(bi, _):
        xb = x_ref[bi]
        h = conv(xb, w1_ref, b1_ref[...])
        h = layer_norm(jnp.maximum(h, 0.0), g1_ref[...], be1_ref[...])
        h = conv(h, w2_ref, b2_ref[...])
        h = layer_norm(jnp.maximum(h, 0.0), g2_ref[...], be2_ref[...])
        o_ref[bi, 0] = ((jnp.sum(h * lw_ref[...], axis=-1) + lb_ref[0, 0])
                        * (1.0 - m_ref[bi, 0]))
        return 0

    lax.fori_loop(0, B, per_batch, 0)


def _predict(x, msk, w1, b1, g1, be1, w2, b2, g2, be2, lw, lb):
    return pl.pallas_call(
        _predictor_body,
        out_shape=jax.ShapeDtypeStruct((B, 1, S), jnp.float32),
    )(x, msk, w1, b1, g1, be1, w2, b2, g2, be2, lw, lb).reshape(B, S)


def kernel(x, src_lens, src_mask, max_len, duration_target,
           conv1_w, conv1_b, ln1_g, ln1_b,
           conv2_w, conv2_b, ln2_g, ln2_b,
           lin_w, lin_b):
    out_flat, tl = _regulate(
        x.reshape(B * S, H), duration_target.astype(jnp.int32),
        src_lens.astype(jnp.int32))

    w1 = jnp.transpose(conv1_w, (2, 1, 0))
    w2 = jnp.transpose(conv2_w, (2, 1, 0))
    pred = _predict(
        x, src_mask.reshape(B, 1, S).astype(jnp.float32),
        w1, conv1_b.reshape(1, F), ln1_g.reshape(1, F),
        ln1_b.reshape(1, F), w2, conv2_b.reshape(1, F),
        ln2_g.reshape(1, F), ln2_b.reshape(1, F),
        lin_w.reshape(1, F), lin_b.reshape(1, 1))

    out = out_flat.reshape(B, MAXLEN, H)
    tgt_len = tl[:, 0]
    return (out, pred, duration_target, tgt_len)


# speedup vs baseline: 1.0383x; 1.0383x over previous
import functools

import jax
import jax.numpy as jnp
from jax import lax
from jax.experimental import pallas as pl
from jax.experimental.pallas import tpu as pltpu
from jax.experimental.pallas import tpu_sc as plsc

B, S, H = 16, 512, 256
F = 256
MAXLEN = 2048

NC, NS = 2, 16
NW = NC * NS
HALF = MAXLEN // NW * NS
ZROW = B * S
CH = 128
NCHUNK = HALF // CH
LANES = 16


def _regulator_kernel(x_hbm, dur_hbm, sl_hbm, out_hbm, tl_hbm,
                      dur_v, sl_v, idx_v, tl_v, buf0, buf1, zbuf,
                      gsem0, gsem1, osem0, osem1, zsem0, zsem1):
    cid = lax.axis_index("c")
    sid = lax.axis_index("s")
    wid = sid * NC + cid
    b = wid // 2
    half = wid % 2

    pltpu.sync_copy(dur_hbm.at[b], dur_v)
    pltpu.sync_copy(sl_hbm, sl_v)

    lane = lax.iota(jnp.int32, LANES)
    bfill = jnp.full((LANES,), b * S, jnp.int32)
    for j in range(NCHUNK):
        for c in range(CH // LANES):
            idx_v[j, pl.ds(c * LANES, LANES)] = bfill

    zvec = jnp.zeros((LANES,), jnp.float32)

    def zrow(i, _):
        for c in range(H // LANES):
            zbuf[i, pl.ds(c * LANES, LANES)] = zvec
        return 0
    lax.fori_loop(0, CH, zrow, 0)

    sl_b = jnp.sum(jnp.where(lane == b, sl_v[...], 0))
    acc = jnp.zeros((LANES,), jnp.int32)
    for c in range(S // LANES):
        t = c * LANES + lane
        acc = acc + jnp.where(t < sl_b, dur_v[pl.ds(c * LANES, LANES)], 0)
    total = jnp.sum(acc)

    out_base = b * MAXLEN
    zsems = (zsem0, zsem1)

    def chunk_start(j):
        return out_base + (2 * j + half) * CH

    def is_real(j):
        return (2 * j + half) * CH < total

    for j in range(NCHUNK):
        @pl.when(jnp.logical_not(is_real(j)))
        def _(j=j):
            pltpu.async_copy(
                zbuf, out_hbm.at[pl.ds(chunk_start(j), CH)], zsems[j % 2])

    carry = jnp.int32(0)
    for c in range(S // LANES):
        t = c * LANES + lane
        d = dur_v[pl.ds(c * LANES, LANES)]
        d = jnp.where(t < sl_b, d, 0)
        cum = plsc.cumsum(d) + carry
        carry = jnp.max(cum)
        prev = cum - d
        gidx = b * S + t
        for r in range(3):
            pos = prev + r
            chunk = lax.shift_right_logical(pos, 7)
            m = (r < d) & (pos < MAXLEN) & ((chunk & 1) == half)
            plsc.store_scatter(
                idx_v,
                [lax.shift_right_logical(pos, 8), pos & (CH - 1)],
                gidx, mask=m)

    @pl.when(half == 0)
    def _():
        tl_v[...] = jnp.full((LANES,), carry, jnp.int32)
        pltpu.sync_copy(tl_v, tl_hbm.at[b])

    bufs = (buf0, buf1)
    gsems = (gsem0, gsem1)
    osems = (osem0, osem1)

    def issue(j):
        p = j % 2

        @pl.when(is_real(j))
        def _():
            pltpu.async_copy(x_hbm.at[idx_v.at[j]], bufs[p], gsems[p])

    def process(j):
        p = j % 2
        lo = (2 * j + half) * CH
        dst = out_hbm.at[pl.ds(chunk_start(j), CH)]

        @pl.when(jnp.logical_not(is_real(j)))
        def _():
            pltpu.make_async_copy(zbuf, dst, zsems[p]).wait()

        @pl.when(is_real(j))
        def _():
            pltpu.make_async_copy(
                x_hbm.at[idx_v.at[j]], bufs[p], gsems[p]).wait()
            nreal = jnp.clip(carry - lo, 0, CH)

            def ztail(i, _):
                for c in range(H // LANES):
                    bufs[p][i, pl.ds(c * LANES, LANES)] = zvec
                return 0
            lax.fori_loop(nreal, CH, ztail, 0)
            pltpu.async_copy(bufs[p], dst, osems[p])

    def drain_out(j):
        p = j % 2

        @pl.when(is_real(j))
        def _():
            pltpu.make_async_copy(
                bufs[p], out_hbm.at[pl.ds(chunk_start(j), CH)],
                osems[p]).wait()

    for j in range(NCHUNK):
        if j >= 2:
            drain_out(j - 2)
        issue(j)
        if j >= 1:
            process(j - 1)
    process(NCHUNK - 1)
    drain_out(NCHUNK - 2)
    drain_out(NCHUNK - 1)


def _regulate(x2d, durations, src_lens):
    mesh = plsc.VectorSubcoreMesh(
        core_axis_name="c", subcore_axis_name="s",
        num_cores=NC, num_subcores=NS)
    run = functools.partial(
        pl.kernel,
        out_type=(
            jax.ShapeDtypeStruct((B * MAXLEN, H), jnp.float32),
            jax.ShapeDtypeStruct((B, LANES), jnp.int32),
        ),
        mesh=mesh,
        scratch_types=[
            pltpu.VMEM((S,), jnp.int32),
            pltpu.VMEM((LANES,), jnp.int32),
            pltpu.VMEM((NCHUNK, CH), jnp.int32),
            pltpu.VMEM((LANES,), jnp.int32),
            pltpu.VMEM((CH, H), jnp.float32),
            pltpu.VMEM((CH, H), jnp.float32),
            pltpu.VMEM((CH, H), jnp.float32),
            pltpu.SemaphoreType.DMA,
            pltpu.SemaphoreType.DMA,
            pltpu.SemaphoreType.DMA,
            pltpu.SemaphoreType.DMA,
            pltpu.SemaphoreType.DMA,
            pltpu.SemaphoreType.DMA,
        ],
        compiler_params=pltpu.CompilerParams(needs_layout_passes=False),
    )(_regulator_kernel)
    return run(x2d, durations, src_lens)


def _predictor_body(x_ref, m_ref, w1_ref, b1_ref, g1_ref, be1_ref,
                    w2_ref, b2_ref, g2_ref, be2_ref, lw_ref, lb_ref, o_ref):
    def conv(h, w_ref, b):
        z = jnp.zeros((1, F), jnp.float32)
        z0 = jnp.dot(h, w_ref[0], preferred_element_type=jnp.float32)
        z1 = jnp.dot(h, w_ref[1], preferred_element_type=jnp.float32)
        z2 = jnp.dot(h, w_ref[2], preferred_element_type=jnp.float32)
        return (jnp.concatenate([z, z0[:-1]], axis=0) + z1 +
                jnp.concatenate([z2[1:], z], axis=0) + b)

    def layer_norm(h, g, be):
        mu = jnp.mean(h, axis=-1, keepdims=True)
        ctr = h - mu
        v = jnp.mean(ctr * ctr, axis=-1, keepdims=True)
        return ctr / jnp.sqrt(v + 1e-5) * g + be

    def per_batch(bi, _):
        xb = x_ref[bi]
        h = conv(xb, w1_ref, b1_ref[...])
        h = layer_norm(jnp.maximum(h, 0.0), g1_ref[...], be1_ref[...])
        h = conv(h, w2_ref, b2_ref[...])
        h = layer_norm(jnp.maximum(h, 0.0), g2_ref[...], be2_ref[...])
        o_ref[bi] = ((jnp.sum(h * lw_ref[...], axis=-1) + lb_ref[0, 0])
                     * (1.0 - m_ref[bi]))
        return 0

    lax.fori_loop(0, B, per_batch, 0)


def _predict(x, msk, w1, b1, g1, be1, w2, b2, g2, be2, lw, lb):
    return pl.pallas_call(
        _predictor_body,
        out_shape=jax.ShapeDtypeStruct((B, S), jnp.float32),
    )(x, msk, w1, b1, g1, be1, w2, b2, g2, be2, lw, lb)


def kernel(x, src_lens, src_mask, max_len, duration_target,
           conv1_w, conv1_b, ln1_g, ln1_b,
           conv2_w, conv2_b, ln2_g, ln2_b,
           lin_w, lin_b):
    out_flat, tl = _regulate(
        x.reshape(B * S, H), duration_target.astype(jnp.int32),
        src_lens.astype(jnp.int32))

    w1 = jnp.transpose(conv1_w, (2, 1, 0))
    w2 = jnp.transpose(conv2_w, (2, 1, 0))
    pred = _predict(
        x, src_mask.astype(jnp.float32),
        w1, conv1_b.reshape(1, F), ln1_g.reshape(1, F),
        ln1_b.reshape(1, F), w2, conv2_b.reshape(1, F),
        ln2_g.reshape(1, F), ln2_b.reshape(1, F),
        lin_w.reshape(1, F), lin_b.reshape(1, 1))

    out = out_flat.reshape(B, MAXLEN, H)
    tgt_len = tl[:, 0]
    return (out, pred, duration_target, tgt_len)


# speedup vs baseline: 1.0434x; 1.0050x over previous
import functools

import jax
import jax.numpy as jnp
from jax import lax
from jax.experimental import pallas as pl
from jax.experimental.pallas import tpu as pltpu
from jax.experimental.pallas import tpu_sc as plsc

B, S, H = 16, 512, 256
F = 256
MAXLEN = 2048

NC, NS = 2, 16
NW = NC * NS
HALF = MAXLEN // NW * NS
ZROW = B * S
CH = 128
NCHUNK = HALF // CH
LANES = 16


def _regulator_kernel(x_hbm, dur_hbm, sl_hbm, out_hbm, tl_hbm,
                      dur_v, sl_v, idx_v, tl_v, buf0, buf1, zbuf,
                      gsem0, gsem1, osem0, osem1, zsem0, zsem1):
    cid = lax.axis_index("c")
    sid = lax.axis_index("s")
    wid = sid * NC + cid
    b = wid // 2
    half = wid % 2

    pltpu.sync_copy(dur_hbm.at[b], dur_v)
    pltpu.sync_copy(sl_hbm, sl_v)

    lane = lax.iota(jnp.int32, LANES)
    bfill = jnp.full((LANES,), b * S, jnp.int32)
    for j in range(NCHUNK):
        for c in range(CH // LANES):
            idx_v[j, pl.ds(c * LANES, LANES)] = bfill

    zvec = jnp.zeros((LANES,), jnp.float32)

    def zrow(i, _):
        for c in range(H // LANES):
            zbuf[i, pl.ds(c * LANES, LANES)] = zvec
        return 0
    lax.fori_loop(0, CH, zrow, 0)

    sl_b = jnp.sum(jnp.where(lane == b, sl_v[...], 0))
    acc = jnp.zeros((LANES,), jnp.int32)
    for c in range(S // LANES):
        t = c * LANES + lane
        acc = acc + jnp.where(t < sl_b, dur_v[pl.ds(c * LANES, LANES)], 0)
    total = jnp.sum(acc)

    out_base = b * MAXLEN
    zsems = (zsem0, zsem1)

    def chunk_start(j):
        return out_base + (2 * j + half) * CH

    def is_real(j):
        return (2 * j + half) * CH < total

    for j in range(NCHUNK):
        @pl.when(jnp.logical_not(is_real(j)))
        def _(j=j):
            pltpu.async_copy(
                zbuf, out_hbm.at[pl.ds(chunk_start(j), CH)], zsems[j % 2])

    carry = jnp.int32(0)
    for c in range(S // LANES):
        t = c * LANES + lane
        d = dur_v[pl.ds(c * LANES, LANES)]
        d = jnp.where(t < sl_b, d, 0)
        cum = plsc.cumsum(d) + carry
        carry = jnp.max(cum)
        prev = cum - d
        gidx = b * S + t
        for r in range(3):
            pos = prev + r
            chunk = lax.shift_right_logical(pos, 7)
            m = (r < d) & (pos < MAXLEN) & ((chunk & 1) == half)
            plsc.store_scatter(
                idx_v,
                [lax.shift_right_logical(pos, 8), pos & (CH - 1)],
                gidx, mask=m)

    @pl.when(half == 1)
    def _():
        tl_v[...] = jnp.full((LANES,), carry, jnp.int32)
        pltpu.sync_copy(tl_v, tl_hbm.at[b])

    bufs = (buf0, buf1)
    gsems = (gsem0, gsem1)
    osems = (osem0, osem1)

    def issue(j):
        p = j % 2

        @pl.when(is_real(j))
        def _():
            pltpu.async_copy(x_hbm.at[idx_v.at[j]], bufs[p], gsems[p])

    def process(j):
        p = j % 2
        lo = (2 * j + half) * CH
        dst = out_hbm.at[pl.ds(chunk_start(j), CH)]

        @pl.when(jnp.logical_not(is_real(j)))
        def _():
            pltpu.make_async_copy(zbuf, dst, zsems[p]).wait()

        @pl.when(is_real(j))
        def _():
            pltpu.make_async_copy(
                x_hbm.at[idx_v.at[j]], bufs[p], gsems[p]).wait()
            nreal = jnp.clip(carry - lo, 0, CH)

            def ztail(i, _):
                for c in range(H // LANES):
                    bufs[p][i, pl.ds(c * LANES, LANES)] = zvec
                return 0
            lax.fori_loop(nreal, CH, ztail, 0)
            pltpu.async_copy(bufs[p], dst, osems[p])

    def drain_out(j):
        p = j % 2

        @pl.when(is_real(j))
        def _():
            pltpu.make_async_copy(
                bufs[p], out_hbm.at[pl.ds(chunk_start(j), CH)],
                osems[p]).wait()

    for j in range(NCHUNK):
        if j >= 2:
            drain_out(j - 2)
        issue(j)
        if j >= 1:
            process(j - 1)
    process(NCHUNK - 1)
    drain_out(NCHUNK - 2)
    drain_out(NCHUNK - 1)


def _regulate(x2d, durations, src_lens):
    mesh = plsc.VectorSubcoreMesh(
        core_axis_name="c", subcore_axis_name="s",
        num_cores=NC, num_subcores=NS)
    run = functools.partial(
        pl.kernel,
        out_type=(
            jax.ShapeDtypeStruct((B * MAXLEN, H), jnp.float32),
            jax.ShapeDtypeStruct((B, LANES), jnp.int32),
        ),
        mesh=mesh,
        scratch_types=[
            pltpu.VMEM((S,), jnp.int32),
            pltpu.VMEM((LANES,), jnp.int32),
            pltpu.VMEM((NCHUNK, CH), jnp.int32),
            pltpu.VMEM((LANES,), jnp.int32),
            pltpu.VMEM((CH, H), jnp.float32),
            pltpu.VMEM((CH, H), jnp.float32),
            pltpu.VMEM((CH, H), jnp.float32),
            pltpu.SemaphoreType.DMA,
            pltpu.SemaphoreType.DMA,
            pltpu.SemaphoreType.DMA,
            pltpu.SemaphoreType.DMA,
            pltpu.SemaphoreType.DMA,
            pltpu.SemaphoreType.DMA,
        ],
        compiler_params=pltpu.CompilerParams(needs_layout_passes=False),
    )(_regulator_kernel)
    return run(x2d, durations, src_lens)


def _predictor_body(x_ref, m_ref, w1_ref, b1_ref, g1_ref, be1_ref,
                    w2_ref, b2_ref, g2_ref, be2_ref, lw_ref, lb_ref, o_ref):
    def conv(h, w_ref, b):
        z = jnp.zeros((1, F), jnp.float32)
        z0 = jnp.dot(h, w_ref[0], preferred_element_type=jnp.float32)
        z1 = jnp.dot(h, w_ref[1], preferred_element_type=jnp.float32)
        z2 = jnp.dot(h, w_ref[2], preferred_element_type=jnp.float32)
        return (jnp.concatenate([z, z0[:-1]], axis=0) + z1 +
                jnp.concatenate([z2[1:], z], axis=0) + b)

    def layer_norm(h, g, be):
        mu = jnp.mean(h, axis=-1, keepdims=True)
        ctr = h - mu
        v = jnp.mean(ctr * ctr, axis=-1, keepdims=True)
        return ctr * lax.rsqrt(v + 1e-5) * g + be

    def per_batch(bi, _):
        xb = x_ref[bi]
        h = conv(xb, w1_ref, b1_ref[...])
        h = layer_norm(jnp.maximum(h, 0.0), g1_ref[...], be1_ref[...])
        h = conv(h, w2_ref, b2_ref[...])
        h = layer_norm(jnp.maximum(h, 0.0), g2_ref[...], be2_ref[...])
        o_ref[bi] = ((jnp.sum(h * lw_ref[...], axis=-1) + lb_ref[0, 0])
                     * (1.0 - m_ref[bi]))
        return 0

    lax.fori_loop(0, B, per_batch, 0)


def _predict(x, msk, w1, b1, g1, be1, w2, b2, g2, be2, lw, lb):
    return pl.pallas_call(
        _predictor_body,
        out_shape=jax.ShapeDtypeStruct((B, S), jnp.float32),
    )(x, msk, w1, b1, g1, be1, w2, b2, g2, be2, lw, lb)


def kernel(x, src_lens, src_mask, max_len, duration_target,
           conv1_w, conv1_b, ln1_g, ln1_b,
           conv2_w, conv2_b, ln2_g, ln2_b,
           lin_w, lin_b):
    out_flat, tl = _regulate(
        x.reshape(B * S, H), duration_target.astype(jnp.int32),
        src_lens.astype(jnp.int32))

    w1 = jnp.transpose(conv1_w, (2, 1, 0))
    w2 = jnp.transpose(conv2_w, (2, 1, 0))
    pred = _predict(
        x, src_mask.astype(jnp.float32),
        w1, conv1_b.reshape(1, F), ln1_g.reshape(1, F),
        ln1_b.reshape(1, F), w2, conv2_b.reshape(1, F),
        ln2_g.reshape(1, F), ln2_b.reshape(1, F),
        lin_w.reshape(1, F), lin_b.reshape(1, 1))

    out = out_flat.reshape(B, MAXLEN, H)
    tgt_len = tl[:, 0]
    return (out, pred, duration_target, tgt_len)


# speedup vs baseline: 1.0490x; 1.0053x over previous
import functools

import jax
import jax.numpy as jnp
from jax import lax
from jax.experimental import pallas as pl
from jax.experimental.pallas import tpu as pltpu
from jax.experimental.pallas import tpu_sc as plsc

B, S, H = 16, 512, 256
F = 256
MAXLEN = 2048

NC, NS = 2, 16
NW = NC * NS
HALF = MAXLEN // NW * NS
ZROW = B * S
CH = 128
NCHUNK = HALF // CH
LANES = 16


def _regulator_kernel(x_hbm, dur_hbm, sl_hbm, out_hbm, tl_hbm,
                      dur_v, sl_v, idx_v, tl_v, buf0, buf1, zbuf,
                      gsem0, gsem1, osem0, osem1, zsem0, zsem1):
    cid = lax.axis_index("c")
    sid = lax.axis_index("s")
    wid = sid * NC + cid
    b = wid // 2
    half = wid % 2

    pltpu.sync_copy(dur_hbm.at[b], dur_v)
    pltpu.sync_copy(sl_hbm, sl_v)

    lane = lax.iota(jnp.int32, LANES)
    bfill = jnp.full((LANES,), b * S, jnp.int32)
    for j in range(NCHUNK):
        for c in range(CH // LANES):
            idx_v[j, pl.ds(c * LANES, LANES)] = bfill

    zvec = jnp.zeros((LANES,), jnp.float32)

    def zrow(i, _):
        for c in range(H // LANES):
            zbuf[i, pl.ds(c * LANES, LANES)] = zvec
        return 0
    lax.fori_loop(0, CH, zrow, 0)

    sl_b = jnp.sum(jnp.where(lane == b, sl_v[...], 0))
    acc = jnp.zeros((LANES,), jnp.int32)
    for c in range(S // LANES):
        t = c * LANES + lane
        acc = acc + jnp.where(t < sl_b, dur_v[pl.ds(c * LANES, LANES)], 0)
    total = jnp.sum(acc)

    out_base = b * MAXLEN
    zsems = (zsem0, zsem1)

    def chunk_start(j):
        return out_base + (2 * j + half) * CH

    def is_real(j):
        return (2 * j + half) * CH < total

    for j in range(NCHUNK):
        @pl.when(jnp.logical_not(is_real(j)))
        def _(j=j):
            pltpu.async_copy(
                zbuf, out_hbm.at[pl.ds(chunk_start(j), CH)], zsems[j % 2])

    carry = jnp.int32(0)
    for c in range(S // LANES):
        t = c * LANES + lane
        d = dur_v[pl.ds(c * LANES, LANES)]
        d = jnp.where(t < sl_b, d, 0)
        cum = plsc.cumsum(d) + carry
        carry = jnp.max(cum)
        prev = cum - d
        gidx = b * S + t
        for r in range(3):
            pos = prev + r
            chunk = lax.shift_right_logical(pos, 7)
            m = (r < d) & (pos < MAXLEN) & ((chunk & 1) == half)
            plsc.store_scatter(
                idx_v,
                [lax.shift_right_logical(pos, 8), pos & (CH - 1)],
                gidx, mask=m)

    @pl.when(half == 1)
    def _():
        tl_v[...] = jnp.full((LANES,), carry, jnp.int32)
        pltpu.sync_copy(tl_v, tl_hbm.at[b])

    bufs = (buf0, buf1)
    gsems = (gsem0, gsem1)
    osems = (osem0, osem1)

    def issue(j):
        p = j % 2

        @pl.when(is_real(j))
        def _():
            pltpu.async_copy(x_hbm.at[idx_v.at[j]], bufs[p], gsems[p])

    def process(j):
        p = j % 2
        lo = (2 * j + half) * CH
        dst = out_hbm.at[pl.ds(chunk_start(j), CH)]

        @pl.when(jnp.logical_not(is_real(j)))
        def _():
            pltpu.make_async_copy(zbuf, dst, zsems[p]).wait()

        @pl.when(is_real(j))
        def _():
            pltpu.make_async_copy(
                x_hbm.at[idx_v.at[j]], bufs[p], gsems[p]).wait()
            nreal = jnp.clip(carry - lo, 0, CH)

            def ztail(i, _):
                for c in range(H // LANES):
                    bufs[p][i, pl.ds(c * LANES, LANES)] = zvec
                return 0
            lax.fori_loop(nreal, CH, ztail, 0)
            pltpu.async_copy(bufs[p], dst, osems[p])

    def drain_out(j):
        p = j % 2

        @pl.when(is_real(j))
        def _():
            pltpu.make_async_copy(
                bufs[p], out_hbm.at[pl.ds(chunk_start(j), CH)],
                osems[p]).wait()

    for j in range(NCHUNK):
        if j >= 2:
            drain_out(j - 2)
        issue(j)
        if j >= 1:
            process(j - 1)
    process(NCHUNK - 1)
    drain_out(NCHUNK - 2)
    drain_out(NCHUNK - 1)


def _regulate(x2d, durations, src_lens):
    mesh = plsc.VectorSubcoreMesh(
        core_axis_name="c", subcore_axis_name="s",
        num_cores=NC, num_subcores=NS)
    run = functools.partial(
        pl.kernel,
        out_type=(
            jax.ShapeDtypeStruct((B * MAXLEN, H), jnp.float32),
            jax.ShapeDtypeStruct((B, LANES), jnp.int32),
        ),
        mesh=mesh,
        scratch_types=[
            pltpu.VMEM((S,), jnp.int32),
            pltpu.VMEM((LANES,), jnp.int32),
            pltpu.VMEM((NCHUNK, CH), jnp.int32),
            pltpu.VMEM((LANES,), jnp.int32),
            pltpu.VMEM((CH, H), jnp.float32),
            pltpu.VMEM((CH, H), jnp.float32),
            pltpu.VMEM((CH, H), jnp.float32),
            pltpu.SemaphoreType.DMA,
            pltpu.SemaphoreType.DMA,
            pltpu.SemaphoreType.DMA,
            pltpu.SemaphoreType.DMA,
            pltpu.SemaphoreType.DMA,
            pltpu.SemaphoreType.DMA,
        ],
        compiler_params=pltpu.CompilerParams(needs_layout_passes=False),
    )(_regulator_kernel)
    return run(x2d, durations, src_lens)


def _predictor_body(x_ref, m_ref, w1_ref, b1_ref, g1_ref, be1_ref,
                    w2_ref, b2_ref, g2_ref, be2_ref, lw_ref, lb_ref, o_ref):
    def conv(h, w_ref, b):
        z = jnp.zeros((1, F), jnp.float32)
        z0 = jnp.dot(h, w_ref[0], preferred_element_type=jnp.float32)
        z1 = jnp.dot(h, w_ref[1], preferred_element_type=jnp.float32)
        z2 = jnp.dot(h, w_ref[2], preferred_element_type=jnp.float32)
        return (jnp.concatenate([z, z0[:-1]], axis=0) + z1 +
                jnp.concatenate([z2[1:], z], axis=0) + b)

    def layer_norm(h, g, be):
        mu = jnp.mean(h, axis=-1, keepdims=True)
        ctr = h - mu
        v = jnp.mean(ctr * ctr, axis=-1, keepdims=True)
        return ctr * lax.rsqrt(v + 1e-5) * g + be

    def per_batch(bi, _):
        xb = x_ref[bi]
        h = conv(xb, w1_ref, b1_ref[...])
        h = layer_norm(jnp.maximum(h, 0.0), g1_ref[...], be1_ref[...])
        h = conv(h, w2_ref, b2_ref[...])
        h = layer_norm(jnp.maximum(h, 0.0), g2_ref[...], be2_ref[...])
        o_ref[bi] = ((jnp.sum(h * lw_ref[...], axis=-1) + lb_ref[0, 0])
                     * (1.0 - m_ref[bi]))
        return 0

    lax.fori_loop(0, B, per_batch, 0)


def _predict(x, msk, w1, b1, g1, be1, w2, b2, g2, be2, lw, lb):
    return pl.pallas_call(
        _predictor_body,
        out_shape=jax.ShapeDtypeStruct((B, S), jnp.float32),
    )(x, msk, w1, b1, g1, be1, w2, b2, g2, be2, lw, lb)


def kernel(x, src_lens, src_mask, max_len, duration_target,
           conv1_w, conv1_b, ln1_g, ln1_b,
           conv2_w, conv2_b, ln2_g, ln2_b,
           lin_w, lin_b):
    out_flat, tl = _regulate(
        x.reshape(B * S, H), duration_target.astype(jnp.int32),
        src_lens.astype(jnp.int32))

    w1 = jnp.transpose(conv1_w, (2, 1, 0))
    w2 = jnp.transpose(conv2_w, (2, 1, 0))
    pred = _predict(
        x, src_mask.astype(jnp.float32),
        w1, conv1_b.reshape(1, F), ln1_g.reshape(1, F),
        ln1_b.reshape(1, F), w2, conv2_b.reshape(1, F),
        ln2_g.reshape(1, F), ln2_b.reshape(1, F),
        lin_w.reshape(1, F), lin_b.reshape(1, 1))

    out = out_flat.reshape(B, MAXLEN, H)
    return (out, pred, duration_target, tl[:, 0])
